# baseline (speedup 1.0000x reference)
import functools

import jax
import jax.numpy as jnp
from jax import lax
from jax.experimental import pallas as pl
from jax.experimental.pallas import tpu as pltpu
from jax.experimental.pallas import tpu_sc as plsc

N_NODES = 10000
N_EDGES = 160000
D_IN = 256
D_OUT = 256
HW = 128
K = 128
NCH = 40
N_TILES = 16
E_PAD = 2 * N_TILES * NCH * K

_sc_mesh = plsc.VectorSubcoreMesh(core_axis_name="c", subcore_axis_name="s")


@functools.partial(
    pl.kernel,
    out_type=jax.ShapeDtypeStruct((2, N_NODES, D_IN), jnp.bfloat16),
    mesh=_sc_mesh,
    compiler_params=pltpu.CompilerParams(
        needs_layout_passes=False, use_tc_tiling_on_sc=False),
    scratch_types=[
        pltpu.VMEM((2, K), jnp.int32),
        pltpu.VMEM((2, K), jnp.int32),
        pltpu.VMEM((NCH, K), jnp.int32),
        pltpu.VMEM((K, HW), jnp.int32),
        pltpu.VMEM((K, D_IN), jnp.bfloat16),
        pltpu.VMEM_SHARED((N_NODES, D_IN), jnp.bfloat16),
        pltpu.SemaphoreType.DMA,
        pltpu.SemaphoreType.DMA,
        pltpu.SemaphoreType.DMA,
    ],
)
def _sc_aggregate(x_hbm, meta_hbm, dst_hbm, z_hbm, out_hbm,
                  mbuf0, mbuf1, dst_v, rows, srows, acc,
                  msem0, msem1, gsem):
    c = lax.axis_index("c")
    s = lax.axis_index("s")

    @pl.when(s == 0)
    def _init():
        pltpu.sync_copy(z_hbm, acc)

    pltpu.sync_copy(dst_hbm.at[c, s], dst_v)

    mbuf = (mbuf0, mbuf1)
    msem = (msem0, msem1)

    def start_meta(k, mb):
        pltpu.async_copy(meta_hbm.at[c, s, k], mbuf[mb], msem[mb])

    def wait_meta(k, mb):
        pltpu.make_async_copy(meta_hbm.at[c, s, k], mbuf[mb], msem[mb]).wait()

    def start_gather(mb):
        pltpu.async_copy(x_hbm.at[mbuf[mb].at[0]], rows, gsem)

    def wait_gather(mb):
        pltpu.make_async_copy(x_hbm.at[mbuf[mb].at[0]], rows, gsem).wait()

    start_meta(0, 0)
    wait_meta(0, 0)
    plsc.subcore_barrier()
    start_gather(0)
    start_meta(1, 1)

    def process(k, mb):
        wait_gather(mb)

        def group_body(g, carry2):
            wv16 = plsc.bitcast(mbuf[mb][1, pl.ds(g * 16, 16)], jnp.float32)
            e0 = g * 16
            for l in range(16):
                wv = lax.gather(
                    wv16, jnp.full((16, 1), l, jnp.int32),
                    dimension_numbers=lax.GatherDimensionNumbers(
                        offset_dims=(), collapsed_slice_dims=(0,),
                        start_index_map=(0,)),
                    slice_sizes=(1,),
                    mode=lax.GatherScatterMode.PROMISE_IN_BOUNDS)
                for j in range(HW // 16):
                    v = plsc.bitcast(
                        rows[e0 + l, pl.ds(j * 16, 16)], jnp.bfloat16)
                    lo, hi = plsc.unpack(v, format=plsc.PackFormat.INTERLEAVED)
                    srows[e0 + l, pl.ds(j * 32, 32)] = plsc.pack(
                        lo * wv, hi * wv, format=plsc.PackFormat.INTERLEAVED)
            return carry2

        lax.fori_loop(0, K // 16, group_body, 0)

        @pl.when(k < NCH - 1)
        def _prefetch():
            wait_meta(k + 1, 1 - mb)
            start_gather(1 - mb)

        @pl.when(k < NCH - 2)
        def _prefetch_meta():
            start_meta(k + 2, mb)

        pltpu.sync_copy(srows, acc.at[dst_v.at[k]], add=True)

    def outer(i, carry):
        process(i * 2, 0)
        process(i * 2 + 1, 1)
        return carry

    lax.fori_loop(0, NCH // 2, outer, 0)
    plsc.subcore_barrier()

    @pl.when(s == 0)
    def _writeback():
        pltpu.sync_copy(acc, out_hbm.at[c])


def _tc_body(agg_ref, w_ref, b_ref, out_ref):
    a = (agg_ref[0].astype(jnp.float32) + agg_ref[1].astype(jnp.float32))
    acc = jnp.dot(a, w_ref[...], preferred_element_type=jnp.float32)
    out_ref[...] = jnp.maximum(acc + b_ref[...], 0.0)


_BM = 2000


@jax.jit
def _tc_matmul(agg, W, b2):
    return pl.pallas_call(
        _tc_body,
        grid=(N_NODES // _BM,),
        in_specs=[
            pl.BlockSpec((2, _BM, D_IN), lambda i: (0, i, 0)),
            pl.BlockSpec((D_IN, D_OUT), lambda i: (0, 0)),
            pl.BlockSpec((1, D_OUT), lambda i: (0, 0)),
        ],
        out_specs=pl.BlockSpec((_BM, D_OUT), lambda i: (i, 0)),
        out_shape=jax.ShapeDtypeStruct((N_NODES, D_OUT), jnp.float32),
    )(agg, W, b2)


def kernel(x, edge_index, edge_weight, W, b):
    dst = edge_index[0].astype(jnp.int32)
    src = edge_index[1].astype(jnp.int32)
    pad = E_PAD - N_EDGES
    zpad = jnp.zeros((pad,), jnp.int32)
    src_p = jnp.concatenate([src, zpad])
    dst_p = jnp.concatenate([dst, zpad])
    w_bits = lax.bitcast_convert_type(
        jnp.concatenate([edge_weight, jnp.zeros((pad,), jnp.float32)]),
        jnp.int32)
    base = jnp.stack([src_p, w_bits])
    meta = jnp.transpose(
        base.reshape(2, 2, N_TILES, NCH, K), (1, 2, 3, 0, 4))
    dst4 = dst_p.reshape(2, N_TILES, NCH, K)
    xp = lax.bitcast_convert_type(
        x.astype(jnp.bfloat16).reshape(N_NODES, HW, 2), jnp.int32)
    z = jnp.zeros((N_NODES, D_IN), jnp.bfloat16)
    agg = _sc_aggregate(xp, meta, dst4, z)
    return _tc_matmul(agg, W, b.reshape(1, D_OUT))


# speedup vs baseline: 1.2491x; 1.2491x over previous
import functools

import jax
import jax.numpy as jnp
from jax import lax
from jax.experimental import pallas as pl
from jax.experimental.pallas import tpu as pltpu
from jax.experimental.pallas import tpu_sc as plsc

N_NODES = 10000
N_EDGES = 160000
D_IN = 256
D_OUT = 256
K = 64
NCH = 80
N_TILES = 16
E_PAD = 2 * N_TILES * NCH * K

_sc_mesh = plsc.VectorSubcoreMesh(core_axis_name="c", subcore_axis_name="s")


@functools.partial(
    pl.kernel,
    out_type=jax.ShapeDtypeStruct((2, N_NODES, D_IN), jnp.bfloat16),
    mesh=_sc_mesh,
    compiler_params=pltpu.CompilerParams(
        needs_layout_passes=False, use_tc_tiling_on_sc=False),
    scratch_types=[
        pltpu.VMEM((2, K), jnp.int32),
        pltpu.VMEM((2, K), jnp.int32),
        pltpu.VMEM((NCH, K), jnp.int32),
        pltpu.VMEM((K, D_IN // 2), jnp.int32),
        pltpu.VMEM((K, D_IN // 2), jnp.int32),
        pltpu.VMEM((K, D_IN), jnp.bfloat16),
        pltpu.VMEM((K, D_IN), jnp.bfloat16),
        pltpu.VMEM_SHARED((N_NODES, D_IN), jnp.bfloat16),
        pltpu.SemaphoreType.DMA,
        pltpu.SemaphoreType.DMA,
        pltpu.SemaphoreType.DMA,
        pltpu.SemaphoreType.DMA,
        pltpu.SemaphoreType.DMA,
        pltpu.SemaphoreType.DMA,
    ],
)
def _sc_aggregate(x_hbm, meta_hbm, dst_hbm, z_hbm, out_hbm,
                  mbuf0, mbuf1, dst_v, rows0, rows1, srows0, srows1, acc,
                  msem0, msem1, gsem0, gsem1, ssem0, ssem1):
    c = lax.axis_index("c")
    s = lax.axis_index("s")

    @pl.when(s == 0)
    def _init():
        pltpu.sync_copy(z_hbm, acc)

    pltpu.sync_copy(dst_hbm.at[c, s], dst_v)

    mbuf = (mbuf0, mbuf1)
    rows = (rows0, rows1)
    srows = (srows0, srows1)
    msem = (msem0, msem1)
    gsem = (gsem0, gsem1)
    ssem = (ssem0, ssem1)

    def start_meta(k, b):
        pltpu.async_copy(meta_hbm.at[c, s, k], mbuf[b], msem[b])

    def wait_meta(k, b):
        pltpu.make_async_copy(meta_hbm.at[c, s, k], mbuf[b], msem[b]).wait()

    def start_gather(b):
        pltpu.async_copy(x_hbm.at[mbuf[b].at[0]], rows[b], gsem[b])

    def wait_gather(b):
        pltpu.make_async_copy(x_hbm.at[mbuf[b].at[0]], rows[b],
                              gsem[b]).wait()

    def wait_scatter(k, b):
        pltpu.make_async_copy(srows[b], acc.at[dst_v.at[k]], ssem[b]).wait()

    start_meta(0, 0)
    wait_meta(0, 0)
    plsc.subcore_barrier()
    start_gather(0)
    start_meta(1, 1)

    def process(k, b):
        nb = 1 - b

        @pl.when(k < NCH - 1)
        def _prefetch():
            @pl.when(k >= 1)
            def _drain():
                wait_scatter(k - 1, nb)

            wait_meta(k + 1, nb)
            start_gather(nb)

        wait_gather(b)

        def group_body(g, carry2):
            wv16 = plsc.bitcast(mbuf[b][1, pl.ds(g * 16, 16)], jnp.float32)
            e0 = g * 16
            for l in range(16):
                wv = lax.gather(
                    wv16, jnp.full((16, 1), l, jnp.int32),
                    dimension_numbers=lax.GatherDimensionNumbers(
                        offset_dims=(), collapsed_slice_dims=(0,),
                        start_index_map=(0,)),
                    slice_sizes=(1,),
                    mode=lax.GatherScatterMode.PROMISE_IN_BOUNDS)
                for j in range(D_IN // 32):
                    v = plsc.bitcast(
                        rows[b][e0 + l, pl.ds(j * 16, 16)], jnp.bfloat16)
                    lo, hi = plsc.unpack(v, format=plsc.PackFormat.INTERLEAVED)
                    srows[b][e0 + l, pl.ds(j * 32, 32)] = plsc.pack(
                        lo * wv, hi * wv, format=plsc.PackFormat.INTERLEAVED)
            return carry2

        lax.fori_loop(0, K // 16, group_body, 0)

        @pl.when(k < NCH - 2)
        def _prefetch_meta():
            start_meta(k + 2, b)

        pltpu.async_copy(srows[b], acc.at[dst_v.at[k]], ssem[b], add=True)

    def outer(i, carry):
        process(i * 2, 0)
        process(i * 2 + 1, 1)
        return carry

    lax.fori_loop(0, NCH // 2, outer, 0)
    wait_scatter(NCH - 2, 0)
    wait_scatter(NCH - 1, 1)
    plsc.subcore_barrier()

    @pl.when(s == 0)
    def _writeback():
        pltpu.sync_copy(acc, out_hbm.at[c])


def _tc_body(agg_ref, w_ref, b_ref, out_ref):
    a = (agg_ref[0].astype(jnp.float32) + agg_ref[1].astype(jnp.float32))
    acc = jnp.dot(a, w_ref[...], preferred_element_type=jnp.float32)
    out_ref[...] = jnp.maximum(acc + b_ref[...], 0.0)


_BM = 2000


@jax.jit
def _tc_matmul(agg, W, b2):
    return pl.pallas_call(
        _tc_body,
        grid=(N_NODES // _BM,),
        in_specs=[
            pl.BlockSpec((2, _BM, D_IN), lambda i: (0, i, 0)),
            pl.BlockSpec((D_IN, D_OUT), lambda i: (0, 0)),
            pl.BlockSpec((1, D_OUT), lambda i: (0, 0)),
        ],
        out_specs=pl.BlockSpec((_BM, D_OUT), lambda i: (i, 0)),
        out_shape=jax.ShapeDtypeStruct((N_NODES, D_OUT), jnp.float32),
    )(agg, W, b2)


def kernel(x, edge_index, edge_weight, W, b):
    dst = edge_index[0].astype(jnp.int32)
    src = edge_index[1].astype(jnp.int32)
    pad = E_PAD - N_EDGES
    zpad = jnp.zeros((pad,), jnp.int32)
    src_p = jnp.concatenate([src, zpad])
    dst_p = jnp.concatenate([dst, zpad])
    w_bits = lax.bitcast_convert_type(
        jnp.concatenate([edge_weight, jnp.zeros((pad,), jnp.float32)]),
        jnp.int32)
    base = jnp.stack([src_p, w_bits])
    meta = jnp.transpose(
        base.reshape(2, 2, N_TILES, NCH, K), (1, 2, 3, 0, 4))
    dst4 = dst_p.reshape(2, N_TILES, NCH, K)
    xb = lax.bitcast_convert_type(
        x.astype(jnp.bfloat16).reshape(N_NODES, D_IN // 2, 2), jnp.int32)
    z = jnp.zeros((N_NODES, D_IN), jnp.bfloat16)
    agg = _sc_aggregate(xb, meta, dst4, z)
    return _tc_matmul(agg, W, b.reshape(1, D_OUT))


# speedup vs baseline: 1.7037x; 1.3640x over previous
import functools

import jax
import jax.numpy as jnp
from jax import lax
from jax.experimental import pallas as pl
from jax.experimental.pallas import tpu as pltpu
from jax.experimental.pallas import tpu_sc as plsc

N_NODES = 10000
N_EDGES = 160000
D_IN = 256
D_OUT = 256
K = 128
NCH = 40
N_TILES = 16
E_PAD = 2 * N_TILES * NCH * K

_sc_mesh = plsc.VectorSubcoreMesh(core_axis_name="c", subcore_axis_name="s")


@functools.partial(
    pl.kernel,
    out_type=jax.ShapeDtypeStruct((2, N_NODES, D_IN), jnp.bfloat16),
    mesh=_sc_mesh,
    compiler_params=pltpu.CompilerParams(
        needs_layout_passes=False, use_tc_tiling_on_sc=False),
    scratch_types=[
        pltpu.VMEM((2, K), jnp.int32),
        pltpu.VMEM((2, K), jnp.int32),
        pltpu.VMEM((NCH, K), jnp.int32),
        pltpu.VMEM((K, D_IN), jnp.bfloat16),
        pltpu.VMEM((K, D_IN), jnp.bfloat16),
        pltpu.VMEM_SHARED((N_NODES, D_IN), jnp.bfloat16),
        pltpu.SemaphoreType.DMA,
        pltpu.SemaphoreType.DMA,
        pltpu.SemaphoreType.DMA,
        pltpu.SemaphoreType.DMA,
        pltpu.SemaphoreType.DMA,
        pltpu.SemaphoreType.DMA,
    ],
)
def _sc_aggregate(x_hbm, meta_hbm, dst_hbm, z_hbm, out_hbm,
                  mbuf0, mbuf1, dst_v, rows0, rows1, acc,
                  msem0, msem1, gsem0, gsem1, ssem0, ssem1):
    c = lax.axis_index("c")
    s = lax.axis_index("s")

    @pl.when(s == 0)
    def _init():
        pltpu.sync_copy(z_hbm, acc)

    pltpu.sync_copy(dst_hbm.at[c, s], dst_v)

    mbuf = (mbuf0, mbuf1)
    rows = (rows0, rows1)
    msem = (msem0, msem1)
    gsem = (gsem0, gsem1)
    ssem = (ssem0, ssem1)

    def start_meta(k, b):
        pltpu.async_copy(meta_hbm.at[c, s, k], mbuf[b], msem[b])

    def wait_meta(k, b):
        pltpu.make_async_copy(meta_hbm.at[c, s, k], mbuf[b], msem[b]).wait()

    def start_gather(b):
        pltpu.async_copy(x_hbm.at[mbuf[b].at[0]], rows[b], gsem[b])

    def wait_gather(b):
        pltpu.make_async_copy(x_hbm.at[mbuf[b].at[0]], rows[b],
                              gsem[b]).wait()

    def wait_scatter(k, b):
        pltpu.make_async_copy(rows[b], acc.at[dst_v.at[k]], ssem[b]).wait()

    start_meta(0, 0)
    wait_meta(0, 0)
    plsc.subcore_barrier()
    start_gather(0)
    start_meta(1, 1)

    def process(k, b):
        nb = 1 - b

        @pl.when(k < NCH - 1)
        def _prefetch():
            @pl.when(k >= 1)
            def _drain():
                wait_scatter(k - 1, nb)

            wait_meta(k + 1, nb)
            start_gather(nb)

        wait_gather(b)

        def group_body(g, carry2):
            wv16 = plsc.bitcast(mbuf[b][1, pl.ds(g * 16, 16)], jnp.float32)
            e0 = g * 16
            for l in range(16):
                wv = lax.gather(
                    wv16, jnp.full((16, 1), l, jnp.int32),
                    dimension_numbers=lax.GatherDimensionNumbers(
                        offset_dims=(), collapsed_slice_dims=(0,),
                        start_index_map=(0,)),
                    slice_sizes=(1,),
                    mode=lax.GatherScatterMode.PROMISE_IN_BOUNDS)
                for j in range(D_IN // 32):
                    sl = pl.ds(j * 32, 32)
                    v = rows[b][e0 + l, sl]
                    lo, hi = plsc.unpack(v, format=plsc.PackFormat.INTERLEAVED)
                    rows[b][e0 + l, sl] = plsc.pack(
                        lo * wv, hi * wv, format=plsc.PackFormat.INTERLEAVED)
            return carry2

        lax.fori_loop(0, K // 16, group_body, 0)

        @pl.when(k < NCH - 2)
        def _prefetch_meta():
            start_meta(k + 2, b)

        pltpu.async_copy(rows[b], acc.at[dst_v.at[k]], ssem[b], add=True)

    def outer(i, carry):
        process(i * 2, 0)
        process(i * 2 + 1, 1)
        return carry

    lax.fori_loop(0, NCH // 2, outer, 0)
    wait_scatter(NCH - 2, 0)
    wait_scatter(NCH - 1, 1)
    plsc.subcore_barrier()

    @pl.when(s == 0)
    def _writeback():
        pltpu.sync_copy(acc, out_hbm.at[c])


def _tc_body(agg_ref, w_ref, b_ref, out_ref):
    a = (agg_ref[0].astype(jnp.float32) + agg_ref[1].astype(jnp.float32))
    acc = jnp.dot(a, w_ref[...], preferred_element_type=jnp.float32)
    out_ref[...] = jnp.maximum(acc + b_ref[...], 0.0)


_BM = 2000


@jax.jit
def _tc_matmul(agg, W, b2):
    return pl.pallas_call(
        _tc_body,
        grid=(N_NODES // _BM,),
        in_specs=[
            pl.BlockSpec((2, _BM, D_IN), lambda i: (0, i, 0)),
            pl.BlockSpec((D_IN, D_OUT), lambda i: (0, 0)),
            pl.BlockSpec((1, D_OUT), lambda i: (0, 0)),
        ],
        out_specs=pl.BlockSpec((_BM, D_OUT), lambda i: (i, 0)),
        out_shape=jax.ShapeDtypeStruct((N_NODES, D_OUT), jnp.float32),
    )(agg, W, b2)


def kernel(x, edge_index, edge_weight, W, b):
    dst = edge_index[0].astype(jnp.int32)
    src = edge_index[1].astype(jnp.int32)
    pad = E_PAD - N_EDGES
    zpad = jnp.zeros((pad,), jnp.int32)
    src_p = jnp.concatenate([src, zpad])
    dst_p = jnp.concatenate([dst, zpad])
    w_bits = lax.bitcast_convert_type(
        jnp.concatenate([edge_weight, jnp.zeros((pad,), jnp.float32)]),
        jnp.int32)
    base = jnp.stack([src_p, w_bits])
    meta = jnp.transpose(
        base.reshape(2, 2, N_TILES, NCH, K), (1, 2, 3, 0, 4))
    dst4 = dst_p.reshape(2, N_TILES, NCH, K)
    xb = x.astype(jnp.bfloat16)
    z = jnp.zeros((N_NODES, D_IN), jnp.bfloat16)
    agg = _sc_aggregate(xb, meta, dst4, z)
    return _tc_matmul(agg, W, b.reshape(1, D_OUT))
